# baseline (speedup 1.0000x reference)
import functools

import jax
import jax.numpy as jnp
from jax import lax
from jax.experimental import pallas as pl
from jax.experimental.pallas import tpu as pltpu
from jax.experimental.pallas import tpu_sc as plsc

TAU = 2.0
GAMMA = 0.7
EPS = 0.05
KS = 1000
KT = 100000
B = 1024

NUM_CORES = 2
NUM_SUBCORES = 16
NUM_WORKERS = NUM_CORES * NUM_SUBCORES
LANES = 16
NVEC = (KS + LANES - 1) // LANES
OUT_ROWS = 1024
J_PER_WORKER = OUT_ROWS // NUM_WORKERS


def _sc_gather_t(teacher_t, t_idx, s_idx):
    mesh = plsc.VectorSubcoreMesh(
        core_axis_name="c", subcore_axis_name="s",
        num_cores=NUM_CORES, num_subcores=NUM_SUBCORES)

    @functools.partial(
        pl.kernel,
        out_type=jax.ShapeDtypeStruct((OUT_ROWS, B), jnp.float32),
        mesh=mesh,
        scratch_types=[
            pltpu.VMEM((KS,), jnp.int32),
            pltpu.VMEM((KS,), jnp.int32),
            pltpu.VMEM((OUT_ROWS,), jnp.int32),
            pltpu.VMEM((J_PER_WORKER, B), jnp.float32),
            pltpu.SemaphoreType.DMA,
        ],
        compiler_params=pltpu.CompilerParams(needs_layout_passes=False),
    )
    def k(teacher_hbm, tidx_hbm, sidx_hbm, out_hbm, tv, sv, cv, slab, sem):
        wid = lax.axis_index("s") * NUM_CORES + lax.axis_index("c")
        pltpu.sync_copy(tidx_hbm, tv)
        pltpu.sync_copy(sidx_hbm, sv)

        pad = jnp.zeros((LANES,), jnp.int32)
        cv[pl.ds(OUT_ROWS - 2 * LANES, LANES)] = pad
        cv[pl.ds(OUT_ROWS - LANES, LANES)] = pad

        def comp(j, carry):
            off = jnp.minimum(j * LANES, KS - LANES)
            plsc.store_scatter(
                cv, [sv[pl.ds(off, LANES)]], tv[pl.ds(off, LANES)])
            return carry
        lax.fori_loop(0, NVEC, comp, 0)

        j0 = wid * J_PER_WORKER

        pltpu.async_copy(
            teacher_hbm.at[cv.at[pl.ds(j0, J_PER_WORKER)]], slab,
            sem).wait()
        pltpu.sync_copy(slab, out_hbm.at[pl.ds(j0, J_PER_WORKER), :])

    return k(teacher_t, t_idx, s_idx)


_BBT = 256


def _tc_student_pre_body(s_ref, msl_ref):
    st = s_ref[...] * (1.0 / TAU)
    ms = jnp.max(st, axis=0, keepdims=True)
    es = jnp.exp(st - ms)
    msl_ref[...] = ms + jnp.log(jnp.sum(es, axis=0, keepdims=True))


def _tc_student_pre(student_t):
    return pl.pallas_call(
        _tc_student_pre_body,
        grid=(B // _BBT,),
        in_specs=[pl.BlockSpec((KS, _BBT), lambda i: (0, i))],
        out_specs=pl.BlockSpec((1, _BBT), lambda i: (0, i)),
        out_shape=jax.ShapeDtypeStruct((1, B), jnp.float32),
    )(student_t)


def _tc_body(g_ref, s_ref, msl_ref, loss_ref, c_ref, w_ref):
    g = g_ref[...] * (1.0 / TAU)
    m = jnp.max(g, axis=0, keepdims=True)
    e = jnp.exp(g - m)
    p = e / jnp.sum(e, axis=0, keepdims=True)
    q = (1.0 - EPS) * p + (EPS / KS)

    st = s_ref[...] * (1.0 / TAU)
    msl = msl_ref[...]

    qc = jnp.maximum(q, 1e-12)
    kl = (jnp.sum(qc * jnp.log(qc), axis=0, keepdims=True)
          - jnp.sum(qc * st, axis=0, keepdims=True)
          + msl * jnp.sum(qc, axis=0, keepdims=True))
    c = jnp.max(q, axis=0, keepdims=True)
    w = jnp.clip((c - GAMMA) / (1.0 - GAMMA), 0.0, 1.0)
    c_ref[...] = c
    w_ref[...] = w

    part = jnp.sum(w * (TAU * TAU) * kl) * (1.0 / B)
    i = pl.program_id(0)

    @pl.when(i == 0)
    def _():
        loss_ref[...] = part[None, None]

    @pl.when(i != 0)
    def _():
        loss_ref[...] += part[None, None]


def _tc_loss(g_t, student_t, msl, interpret=False):
    return pl.pallas_call(
        _tc_body,
        grid=(B // _BBT,),
        in_specs=[
            pl.BlockSpec((KS, _BBT), lambda i: (0, i)),
            pl.BlockSpec((KS, _BBT), lambda i: (0, i)),
            pl.BlockSpec((1, _BBT), lambda i: (0, i)),
        ],
        out_specs=[
            pl.BlockSpec((1, 1), lambda i: (0, 0)),
            pl.BlockSpec((1, _BBT), lambda i: (0, i)),
            pl.BlockSpec((1, _BBT), lambda i: (0, i)),
        ],
        out_shape=[
            jax.ShapeDtypeStruct((1, 1), jnp.float32),
            jax.ShapeDtypeStruct((1, B), jnp.float32),
            jax.ShapeDtypeStruct((1, B), jnp.float32),
        ],
        interpret=interpret,
    )(g_t, student_t, msl)


def kernel(teacher_logits_w, student_logits_s, t_idx, s_idx):
    t_idx = jnp.asarray(t_idx, jnp.int32)
    s_idx = jnp.asarray(s_idx, jnp.int32)
    g_t = _sc_gather_t(teacher_logits_w.T, t_idx, s_idx)
    msl = _tc_student_pre(student_logits_s.T)
    loss2, c2, w2 = _tc_loss(g_t, student_logits_s.T, msl)
    return (loss2[0, 0],
            jax.lax.stop_gradient(c2[0]),
            jax.lax.stop_gradient(w2[0]))


# speedup vs baseline: 1.0480x; 1.0480x over previous
import functools

import jax
import jax.numpy as jnp
from jax import lax
from jax.experimental import pallas as pl
from jax.experimental.pallas import tpu as pltpu
from jax.experimental.pallas import tpu_sc as plsc

TAU = 2.0
GAMMA = 0.7
EPS = 0.05
KS = 1000
KT = 100000
B = 1024

NUM_CORES = 2
NUM_SUBCORES = 16
NUM_WORKERS = NUM_CORES * NUM_SUBCORES
LANES = 16
NVEC = (KS + LANES - 1) // LANES
OUT_ROWS = 1024
J_PER_WORKER = OUT_ROWS // NUM_WORKERS


def _sc_gather_t(teacher_t, t_idx, s_idx):
    mesh = plsc.VectorSubcoreMesh(
        core_axis_name="c", subcore_axis_name="s",
        num_cores=NUM_CORES, num_subcores=NUM_SUBCORES)

    @functools.partial(
        pl.kernel,
        out_type=jax.ShapeDtypeStruct((OUT_ROWS, B), jnp.float32),
        mesh=mesh,
        scratch_types=[
            pltpu.VMEM((KS,), jnp.int32),
            pltpu.VMEM((KS,), jnp.int32),
            pltpu.VMEM((OUT_ROWS,), jnp.int32),
            pltpu.VMEM((J_PER_WORKER, B), jnp.float32),
            pltpu.SemaphoreType.DMA,
            pltpu.SemaphoreType.DMA,
        ],
        compiler_params=pltpu.CompilerParams(needs_layout_passes=False),
    )
    def k(teacher_hbm, tidx_hbm, sidx_hbm, out_hbm, tv, sv, cv, slab, sem,
          sem2):
        wid = lax.axis_index("s") * NUM_CORES + lax.axis_index("c")
        pltpu.async_copy(tidx_hbm, tv, sem)
        pltpu.async_copy(sidx_hbm, sv, sem2)
        pltpu.make_async_copy(tidx_hbm, tv, sem).wait()
        pltpu.make_async_copy(sidx_hbm, sv, sem2).wait()

        pad = jnp.zeros((LANES,), jnp.int32)
        cv[pl.ds(OUT_ROWS - 2 * LANES, LANES)] = pad
        cv[pl.ds(OUT_ROWS - LANES, LANES)] = pad

        def comp(j, carry):
            off = jnp.minimum(j * LANES, KS - LANES)
            plsc.store_scatter(
                cv, [sv[pl.ds(off, LANES)]], tv[pl.ds(off, LANES)])
            return carry
        lax.fori_loop(0, NVEC, comp, 0)

        j0 = wid * J_PER_WORKER
        half = J_PER_WORKER // 2

        g0 = pltpu.async_copy(
            teacher_hbm.at[cv.at[pl.ds(j0, half)]],
            slab.at[pl.ds(0, half), :], sem)
        g1 = pltpu.async_copy(
            teacher_hbm.at[cv.at[pl.ds(j0 + half, half)]],
            slab.at[pl.ds(half, half), :], sem2)
        g0.wait()
        w0 = pltpu.async_copy(
            slab.at[pl.ds(0, half), :], out_hbm.at[pl.ds(j0, half), :], sem)
        g1.wait()
        w1 = pltpu.async_copy(
            slab.at[pl.ds(half, half), :],
            out_hbm.at[pl.ds(j0 + half, half), :], sem2)
        w0.wait()
        w1.wait()

    return k(teacher_t, t_idx, s_idx)


_BBT = 256


def _tc_body(g_ref, s_ref, loss_ref, c_ref, w_ref):
    g = g_ref[...] * (1.0 / TAU)
    m = jnp.max(g, axis=0, keepdims=True)
    e = jnp.exp(g - m)
    p = e / jnp.sum(e, axis=0, keepdims=True)
    q = (1.0 - EPS) * p + (EPS / KS)

    st = s_ref[...] * (1.0 / TAU)
    ms = jnp.max(st, axis=0, keepdims=True)
    es = jnp.exp(st - ms)
    msl = ms + jnp.log(jnp.sum(es, axis=0, keepdims=True))

    qc = jnp.maximum(q, 1e-12)
    kl = (jnp.sum(qc * jnp.log(qc), axis=0, keepdims=True)
          - jnp.sum(qc * st, axis=0, keepdims=True)
          + msl * jnp.sum(qc, axis=0, keepdims=True))
    c = jnp.max(q, axis=0, keepdims=True)
    w = jnp.clip((c - GAMMA) / (1.0 - GAMMA), 0.0, 1.0)
    c_ref[...] = c
    w_ref[...] = w

    part = jnp.sum(w * (TAU * TAU) * kl) * (1.0 / B)
    i = pl.program_id(0)

    @pl.when(i == 0)
    def _():
        loss_ref[...] = part[None, None]

    @pl.when(i != 0)
    def _():
        loss_ref[...] += part[None, None]


def _tc_loss(g_t, student_t, interpret=False):
    return pl.pallas_call(
        _tc_body,
        grid=(B // _BBT,),
        in_specs=[
            pl.BlockSpec((KS, _BBT), lambda i: (0, i)),
            pl.BlockSpec((KS, _BBT), lambda i: (0, i)),
        ],
        out_specs=[
            pl.BlockSpec((1, 1), lambda i: (0, 0)),
            pl.BlockSpec((1, _BBT), lambda i: (0, i)),
            pl.BlockSpec((1, _BBT), lambda i: (0, i)),
        ],
        out_shape=[
            jax.ShapeDtypeStruct((1, 1), jnp.float32),
            jax.ShapeDtypeStruct((1, B), jnp.float32),
            jax.ShapeDtypeStruct((1, B), jnp.float32),
        ],
        interpret=interpret,
    )(g_t, student_t)


def kernel(teacher_logits_w, student_logits_s, t_idx, s_idx):
    t_idx = jnp.asarray(t_idx, jnp.int32)
    s_idx = jnp.asarray(s_idx, jnp.int32)
    g_t = _sc_gather_t(teacher_logits_w.T, t_idx, s_idx)
    loss2, c2, w2 = _tc_loss(g_t, student_logits_s.T)
    return (loss2[0, 0],
            jax.lax.stop_gradient(c2[0]),
            jax.lax.stop_gradient(w2[0]))
